# baseline (speedup 1.0000x reference)
import functools

import jax
import jax.numpy as jnp
from jax import lax
from jax.experimental import pallas as pl
from jax.experimental.pallas import tpu as pltpu
from jax.experimental.pallas import tpu_sc as plsc


def _tc_stream_body(nsteps, U, x_ref, pu_ref, ut_ref, pi_ref, it_ref,
                    ucomb_ref, icomb_ref):
    i = pl.program_id(0)
    BLK = x_ref.shape[0]
    x = x_ref[...]
    valid = U - i * BLK
    rowmask = lax.broadcasted_iota(jnp.int32, (BLK, 1), 0) < valid
    x = jnp.where(rowmask, x, 0.0)
    pu = jnp.where(rowmask, pu_ref[...], 0.0)

    ucomb_ref[...] = (
        jax.lax.dot_general(x, pi_ref[...], (((1,), (0,)), ((), ())),
                            preferred_element_type=jnp.float32)
        + ut_ref[...]
    )

    @pl.when(i == 0)
    def _():
        icomb_ref[...] = jnp.zeros_like(icomb_ref)

    icomb_ref[...] += jax.lax.dot_general(
        x, pu, (((0,), (0,)), ((), ())), preferred_element_type=jnp.float32)

    @pl.when(i == nsteps - 1)
    def _():
        icomb_ref[...] += it_ref[...]


def _tc_stream(interactions, P_user, user_table, P_item, item_table, BLK=1024):
    U, I = interactions.shape
    K = P_item.shape[1]
    nsteps = pl.cdiv(U, BLK)
    return pl.pallas_call(
        functools.partial(_tc_stream_body, nsteps, U),
        grid=(nsteps,),
        in_specs=[
            pl.BlockSpec((BLK, I), lambda i: (i, 0)),
            pl.BlockSpec((BLK, K), lambda i: (i, 0)),
            pl.BlockSpec((BLK, K), lambda i: (i, 0)),
            pl.BlockSpec((I, K), lambda i: (0, 0)),
            pl.BlockSpec((I, K), lambda i: (0, 0)),
        ],
        out_specs=[
            pl.BlockSpec((BLK, K), lambda i: (i, 0)),
            pl.BlockSpec((I, K), lambda i: (0, 0)),
        ],
        out_shape=[
            jax.ShapeDtypeStruct((U, K), jnp.float32),
            jax.ShapeDtypeStruct((I, K), jnp.float32),
        ],
        compiler_params=pltpu.CompilerParams(
            dimension_semantics=("arbitrary",),
        ),
    )(interactions, P_user, user_table, P_item, item_table)


def _sc_gather_mul(user_idx, item_idx, U_comb, I_comb):
    B = user_idx.shape[0]
    K = U_comb.shape[1]
    info = plsc.get_sparse_core_info()
    NC, NS, L = info.num_cores, info.num_subcores, info.num_lanes
    NW = NC * NS
    assert B % NW == 0
    b_per_w = B // NW
    mesh = plsc.VectorSubcoreMesh(core_axis_name="c", subcore_axis_name="s")

    @functools.partial(
        pl.kernel,
        mesh=mesh,
        out_type=jax.ShapeDtypeStruct((B, K), jnp.float32),
        scratch_types=[
            pltpu.VMEM((b_per_w,), jnp.int32),
            pltpu.VMEM((b_per_w,), jnp.int32),
            pltpu.VMEM((b_per_w, K), jnp.float32),
            pltpu.VMEM((b_per_w, K), jnp.float32),
            pltpu.VMEM((b_per_w, K), jnp.float32),
            pltpu.SemaphoreType.DMA,
            pltpu.SemaphoreType.DMA,
        ],
        compiler_params=pltpu.CompilerParams(use_tc_tiling_on_sc=False),
    )
    def sc_k(uidx_hbm, iidx_hbm, ucomb_hbm, icomb_hbm, out_hbm,
             uidx_v, iidx_v, urows_v, irows_v, out_v, sem_u, sem_i):
        wid = lax.axis_index("s") * NC + lax.axis_index("c")
        base = wid * b_per_w
        pltpu.sync_copy(uidx_hbm.at[pl.ds(base, b_per_w)], uidx_v)
        pltpu.sync_copy(iidx_hbm.at[pl.ds(base, b_per_w)], iidx_v)
        cp_u = pltpu.async_copy(ucomb_hbm.at[uidx_v], urows_v, sem_u)
        cp_i = pltpu.async_copy(icomb_hbm.at[iidx_v], irows_v, sem_i)
        cp_u.wait()
        cp_i.wait()

        def body(r, carry):
            for h in range(K // L):
                sl = pl.ds(h * L, L)
                out_v[r, sl] = urows_v[r, sl] * irows_v[r, sl]
            return carry

        lax.fori_loop(0, b_per_w, body, 0)
        pltpu.sync_copy(out_v, out_hbm.at[pl.ds(base, b_per_w)])

    return sc_k(user_idx, item_idx, U_comb, I_comb)


def kernel(user_idx, item_idx, interactions, user_table, item_table,
           P_user, P_item):
    U_comb, I_comb = _tc_stream(interactions, P_user, user_table,
                                P_item, item_table)
    return _sc_gather_mul(user_idx.astype(jnp.int32),
                          item_idx.astype(jnp.int32), U_comb, I_comb)
